# Initial kernel scaffold; baseline (speedup 1.0000x reference)
import jax
import jax.numpy as jnp
from jax.experimental import pallas as pl


def kernel(x, x_c, adj, W_g, W1, b1, W2, b2, Wf, bf):
    raise NotImplementedError("write your pallas kernel here")


# speedup vs baseline: 1.2591x; 1.2591x over previous
import jax
import jax.numpy as jnp
from jax import lax
from jax.experimental import pallas as pl
from jax.experimental.pallas import tpu as pltpu

B, N, D, H = 8, 512, 768, 2048
TOPK = 180
NKEEP = 506
KPAD = 192
PAD_IDX = 511


def _score_topk_body(xc_ref, score_ref, idx_ref, keep_ref, sel_ref):
    x = xc_ref[...]
    norm2 = jnp.sum(x * x, axis=2)
    nrm = jnp.maximum(jnp.sqrt(norm2), 1e-12)
    xn = x / nrm[:, :, None]
    lm = x[:, N - 1, :]
    l2 = jnp.sum(lm * lm, axis=1)
    ln = (lm / jnp.maximum(jnp.sqrt(l2), 1e-12)[:, None])[:, None, :]
    sen_doc = lax.dot_general(
        xn.astype(jnp.bfloat16), ln.astype(jnp.bfloat16),
        (((2,), (2,)), ((0,), (0,))),
        preferred_element_type=jnp.float32)[..., 0]
    sen_doc = jnp.where(sen_doc > 0, sen_doc, 0.0)
    g = lax.dot_general(xn, xn, (((2,), (2,)), ((0,), (0,))),
                        preferred_element_type=jnp.float32)
    sen_sen = jnp.mean(g, axis=-1)
    sen_sen = jnp.where(sen_sen > 0, sen_sen, 1.0)
    score = sen_doc * 0.9 + (1.0 - sen_sen) * 0.1
    score_ref[...] = score

    pos = lax.broadcasted_iota(jnp.int32, (B, N), 1)
    lanes = lax.broadcasted_iota(jnp.int32, (B, KPAD), 1)
    neg_inf = jnp.float32(-jnp.inf)
    work0 = jnp.where(pos < NKEEP, score, neg_inf)
    idx0 = jnp.full((B, KPAD), PAD_IDX, jnp.int32)

    def body(k, carry):
        work, idxv = carry
        m = jnp.max(work, axis=1, keepdims=True)
        cand = jnp.where(work == m, pos, N)
        i = jnp.min(cand, axis=1, keepdims=True)
        idxv = jnp.where(lanes == k, i, idxv)
        work = jnp.where(pos == i, neg_inf, work)
        return work, idxv

    work, idxv = lax.fori_loop(0, TOPK, body, (work0, idx0))
    idx_ref[...] = idxv
    selected = (work == neg_inf) & (pos < NKEEP)
    sel_ref[...] = selected.astype(jnp.float32)
    keep_ref[...] = (selected | (pos >= NKEEP)).astype(jnp.float32)


def _gcn_mlp_body(xc_ref, adj_ref, idx_ref, keep_ref, sel_ref,
                  wg_ref, w1_ref, b1_ref, w2_ref, b2_ref, wf_ref, bf_ref,
                  out_ref, xmask_ref):
    f32 = jnp.float32
    X = xc_ref[0]
    A = adj_ref[0]
    idxr = idx_ref[0, 0, :]
    keepr = keep_ref[0]
    selr = sel_ref[0]

    idc = idxr.reshape(KPAD, 1)
    colp = lax.broadcasted_iota(jnp.int32, (KPAD, N), 1)
    G = (colp == idc).astype(f32)

    A_r = jnp.dot(G, A, preferred_element_type=f32)
    A_rm = A_r * keepr
    P = jnp.dot(A_rm, X, preferred_element_type=f32)
    h = jnp.maximum(jnp.dot(P, wg_ref[...], preferred_element_type=f32), 0.0)
    t = jnp.dot(h, w1_ref[...], preferred_element_type=f32) + b1_ref[...]
    t = jnp.where(t >= 0, t, 0.01 * t)
    u = jnp.dot(t, w2_ref[...], preferred_element_type=f32) + b2_ref[...]
    bf = bf_ref[0, 0]
    oc = jnp.sum(u * wf_ref[...], axis=1) + bf

    z1 = b1_ref[...]
    z1 = jnp.where(z1 >= 0, z1, 0.01 * z1)
    z2 = jnp.dot(z1, w2_ref[...], preferred_element_type=f32) + b2_ref[...]
    zo = jnp.sum(z2 * wf_ref[...]) + bf

    outrow = jnp.dot(oc.reshape(1, KPAD), G, preferred_element_type=f32)
    out_ref[0] = outrow + (1.0 - selr) * zo

    xmask_ref[0] = jnp.broadcast_to(keepr.reshape(N, 1), (N, D))


def kernel(x, x_c, adj, W_g, W1, b1, W2, b2, Wf, bf):
    f32 = jnp.float32
    score, idx, keep, sel = pl.pallas_call(
        _score_topk_body,
        out_shape=(
            jax.ShapeDtypeStruct((B, N), f32),
            jax.ShapeDtypeStruct((B, KPAD), jnp.int32),
            jax.ShapeDtypeStruct((B, N), f32),
            jax.ShapeDtypeStruct((B, N), f32),
        ),
    )(x_c)

    idx3 = idx.reshape(B, 1, KPAD)
    keep3 = keep.reshape(B, 1, N)
    sel3 = sel.reshape(B, 1, N)
    b1r = b1.reshape(1, H)
    b2r = b2.reshape(1, D)
    wfr = Wf.reshape(1, D)
    bfr = bf.reshape(1, 1)

    out_full, x_mask = pl.pallas_call(
        _gcn_mlp_body,
        grid=(B,),
        in_specs=[
            pl.BlockSpec((1, N, D), lambda b: (b, 0, 0)),
            pl.BlockSpec((1, N, N), lambda b: (b, 0, 0)),
            pl.BlockSpec((1, 1, KPAD), lambda b: (b, 0, 0)),
            pl.BlockSpec((1, 1, N), lambda b: (b, 0, 0)),
            pl.BlockSpec((1, 1, N), lambda b: (b, 0, 0)),
            pl.BlockSpec((D, D), lambda b: (0, 0)),
            pl.BlockSpec((D, H), lambda b: (0, 0)),
            pl.BlockSpec((1, H), lambda b: (0, 0)),
            pl.BlockSpec((H, D), lambda b: (0, 0)),
            pl.BlockSpec((1, D), lambda b: (0, 0)),
            pl.BlockSpec((1, D), lambda b: (0, 0)),
            pl.BlockSpec((1, 1), lambda b: (0, 0)),
        ],
        out_specs=(
            pl.BlockSpec((1, 1, N), lambda b: (b, 0, 0)),
            pl.BlockSpec((1, N, D), lambda b: (b, 0, 0)),
        ),
        out_shape=(
            jax.ShapeDtypeStruct((B, 1, N), f32),
            jax.ShapeDtypeStruct((B, N, D), f32),
        ),
    )(x_c, adj, idx3, keep3, sel3, W_g, W1, b1r, W2, b2r, wfr, bfr)

    out = out_full[:, 0, :NKEEP, None]
    selected_idx = idx[:, :TOPK]
    score_leaf = score[:, :NKEEP, None]
    return out, selected_idx, score_leaf, x_mask


# speedup vs baseline: 1.7734x; 1.4085x over previous
import functools

import jax
import jax.numpy as jnp
from jax import lax
from jax.experimental import pallas as pl
from jax.experimental.pallas import tpu as pltpu
from jax.experimental.pallas import tpu_sc as plsc

B, N, D, H = 8, 512, 768, 2048
TOPK = 180
NKEEP = 506
KPAD = 192
PAD_IDX = 511
NC, NS = 2, 16
ROWS_PER_TILE = B * KPAD // (NC * NS)


def _score_topk_body(xc_ref, score_ref, idx_ref, idxflat_ref, keep_ref, sel_ref):
    x = xc_ref[...]
    norm2 = jnp.sum(x * x, axis=2)
    nrm = jnp.maximum(jnp.sqrt(norm2), 1e-12)
    xn = x / nrm[:, :, None]
    lm = x[:, N - 1, :]
    l2 = jnp.sum(lm * lm, axis=1)
    ln = (lm / jnp.maximum(jnp.sqrt(l2), 1e-12)[:, None])[:, None, :]
    sen_doc = lax.dot_general(
        xn.astype(jnp.bfloat16), ln.astype(jnp.bfloat16),
        (((2,), (2,)), ((0,), (0,))),
        preferred_element_type=jnp.float32)[..., 0]
    sen_doc = jnp.where(sen_doc > 0, sen_doc, 0.0)
    g = lax.dot_general(xn, xn, (((2,), (2,)), ((0,), (0,))),
                        preferred_element_type=jnp.float32)
    sen_sen = jnp.mean(g, axis=-1)
    sen_sen = jnp.where(sen_sen > 0, sen_sen, 1.0)
    score = sen_doc * 0.9 + (1.0 - sen_sen) * 0.1
    score_ref[...] = score

    pos = lax.broadcasted_iota(jnp.int32, (B, N), 1)
    neg_inf = jnp.float32(-jnp.inf)
    work = jnp.where(pos < NKEEP, score, neg_inf)

    s_col = work.reshape(B, N, 1)
    s_row = work.reshape(B, 1, N)
    ic = lax.broadcasted_iota(jnp.int32, (B, N, 1), 1)
    ir = lax.broadcasted_iota(jnp.int32, (B, 1, N), 2)
    beats = (s_col > s_row) | ((s_col == s_row) & (ic < ir))
    rank = jnp.sum(beats.astype(jnp.float32), axis=1)

    selected = rank < float(TOPK)
    sel_ref[...] = selected.astype(jnp.float32)
    keep_ref[...] = (selected | (pos >= NKEEP)).astype(jnp.float32)

    rank_col = rank.reshape(B, N, 1).astype(jnp.int32)
    rcols = lax.broadcasted_iota(jnp.int32, (B, 1, KPAD), 2)
    onehot = ((rank_col == rcols) &
              (rcols < TOPK)).astype(jnp.float32)
    posf = lax.broadcasted_iota(jnp.int32, (B, N, 1), 1).astype(jnp.float32)
    idx_vals = jnp.sum(posf * onehot, axis=1)
    hit = jnp.sum(onehot, axis=1)
    idxv = (idx_vals + (1.0 - hit) * PAD_IDX).astype(jnp.int32)
    idx_ref[...] = idxv
    brow = lax.broadcasted_iota(jnp.int32, (B, KPAD), 0)
    idxflat_ref[...] = idxv + brow * N


def _sc_gather_body(adj_hbm, idx_hbm, out_hbm, idx_v, rows_v, sem):
    wid = lax.axis_index("s") * NC + lax.axis_index("c")
    base = wid * ROWS_PER_TILE
    pltpu.sync_copy(idx_hbm.at[pl.ds(base, ROWS_PER_TILE)], idx_v)
    pltpu.async_copy(adj_hbm.at[idx_v], rows_v, sem).wait()
    pltpu.sync_copy(rows_v, out_hbm.at[pl.ds(base, ROWS_PER_TILE)])


def _gcn_mlp_body(xc_ref, ar_ref, idx_ref, keep_ref, sel_ref,
                  wg_ref, w1_ref, b1_ref, w2_ref, b2_ref, wf_ref, bf_ref,
                  out_ref, xmask_ref):
    f32 = jnp.float32
    X = xc_ref[0]
    A_r = ar_ref[0]
    idxr = idx_ref[0, 0, :]
    keepr = keep_ref[0]
    selr = sel_ref[0]

    idc = idxr.reshape(KPAD, 1)
    colp = lax.broadcasted_iota(jnp.int32, (KPAD, N), 1)
    G = (colp == idc).astype(f32)

    A_rm = A_r * keepr
    P = jnp.dot(A_rm, X, preferred_element_type=f32)
    h = jnp.maximum(jnp.dot(P, wg_ref[...], preferred_element_type=f32), 0.0)
    t = jnp.dot(h, w1_ref[...], preferred_element_type=f32) + b1_ref[...]
    t = jnp.where(t >= 0, t, 0.01 * t)
    u = jnp.dot(t, w2_ref[...], preferred_element_type=f32) + b2_ref[...]
    bf = bf_ref[0, 0]
    oc = jnp.sum(u * wf_ref[...], axis=1) + bf

    z1 = b1_ref[...]
    z1 = jnp.where(z1 >= 0, z1, 0.01 * z1)
    z2 = jnp.dot(z1, w2_ref[...], preferred_element_type=f32) + b2_ref[...]
    zo = jnp.sum(z2 * wf_ref[...]) + bf

    outrow = jnp.dot(oc.reshape(1, KPAD), G, preferred_element_type=f32)
    out_ref[0] = outrow + (1.0 - selr) * zo

    xmask_ref[0] = jnp.broadcast_to(keepr.reshape(N, 1), (N, D))


def kernel(x, x_c, adj, W_g, W1, b1, W2, b2, Wf, bf):
    f32 = jnp.float32
    score, idx, idxflat, keep, sel = pl.pallas_call(
        _score_topk_body,
        out_shape=(
            jax.ShapeDtypeStruct((B, N), f32),
            jax.ShapeDtypeStruct((B, KPAD), jnp.int32),
            jax.ShapeDtypeStruct((B, KPAD), jnp.int32),
            jax.ShapeDtypeStruct((B, N), f32),
            jax.ShapeDtypeStruct((B, N), f32),
        ),
    )(x_c)

    sc_gather = functools.partial(
        pl.kernel,
        mesh=plsc.VectorSubcoreMesh(core_axis_name="c", subcore_axis_name="s"),
        out_type=jax.ShapeDtypeStruct((B * KPAD, N), f32),
        scratch_types=[
            pltpu.VMEM((ROWS_PER_TILE,), jnp.int32),
            pltpu.VMEM((ROWS_PER_TILE, N), f32),
            pltpu.SemaphoreType.DMA,
        ],
    )(_sc_gather_body)
    a_rows = sc_gather(adj.reshape(B * N, N), idxflat.reshape(B * KPAD))
    a_rows = a_rows.reshape(B, KPAD, N)

    idx3 = idx.reshape(B, 1, KPAD)
    keep3 = keep.reshape(B, 1, N)
    sel3 = sel.reshape(B, 1, N)
    b1r = b1.reshape(1, H)
    b2r = b2.reshape(1, D)
    wfr = Wf.reshape(1, D)
    bfr = bf.reshape(1, 1)

    out_full, x_mask = pl.pallas_call(
        _gcn_mlp_body,
        grid=(B,),
        in_specs=[
            pl.BlockSpec((1, N, D), lambda b: (b, 0, 0)),
            pl.BlockSpec((1, KPAD, N), lambda b: (b, 0, 0)),
            pl.BlockSpec((1, 1, KPAD), lambda b: (b, 0, 0)),
            pl.BlockSpec((1, 1, N), lambda b: (b, 0, 0)),
            pl.BlockSpec((1, 1, N), lambda b: (b, 0, 0)),
            pl.BlockSpec((D, D), lambda b: (0, 0)),
            pl.BlockSpec((D, H), lambda b: (0, 0)),
            pl.BlockSpec((1, H), lambda b: (0, 0)),
            pl.BlockSpec((H, D), lambda b: (0, 0)),
            pl.BlockSpec((1, D), lambda b: (0, 0)),
            pl.BlockSpec((1, D), lambda b: (0, 0)),
            pl.BlockSpec((1, 1), lambda b: (0, 0)),
        ],
        out_specs=(
            pl.BlockSpec((1, 1, N), lambda b: (b, 0, 0)),
            pl.BlockSpec((1, N, D), lambda b: (b, 0, 0)),
        ),
        out_shape=(
            jax.ShapeDtypeStruct((B, 1, N), f32),
            jax.ShapeDtypeStruct((B, N, D), f32),
        ),
    )(x_c, a_rows, idx3, keep3, sel3, W_g, W1, b1r, W2, b2r, wfr, bfr)

    out = out_full[:, 0, :NKEEP, None]
    selected_idx = idx[:, :TOPK]
    score_leaf = score[:, :NKEEP, None]
    return out, selected_idx, score_leaf, x_mask
